# baseline (speedup 1.0000x reference)
import functools

import jax
import jax.numpy as jnp
from jax import lax
from jax.experimental import pallas as pl
from jax.experimental.pallas import tpu as pltpu

B, P, D = 16, 4096, 128
INTER, POOL, OUT = 64, 64, 32
K = 16
GB = 4

_SLOPE = 0.01


def _leaky(v):
    return jnp.where(v >= 0, v, _SLOPE * v)


def _fused_body(x_ref, w1, b1, w2, b2, w3, b3, w4, b4, w5, b5, w6, b6, cen,
                assign_ref, y_ref, seg):
    g = pl.program_id(0)
    c = cen[...]
    cn = jnp.sum(c * c, axis=1, keepdims=True)

    for bi in range(GB):
        xb = x_ref[bi]
        h = _leaky(jnp.dot(xb, w1[...], preferred_element_type=jnp.float32) + b1[...])
        xc = jnp.dot(h, w2[...], preferred_element_type=jnp.float32) + b2[...]
        cross_t = lax.dot_general(c, xc, (((1,), (1,)), ((), ())),
                                  preferred_element_type=jnp.float32)
        score = cn - 2.0 * cross_t
        best = jnp.min(score, axis=0, keepdims=True)
        ii = lax.broadcasted_iota(jnp.int32, (K, P), 0)
        a2d = jnp.min(jnp.where(score == best, ii, K), axis=0, keepdims=True)
        assign_ref[bi] = a2d

        oh_t = (ii == a2d).astype(jnp.float32)
        xc1 = jnp.concatenate([xc, jnp.ones((P, 1), jnp.float32)], axis=1)
        seg[pl.ds((g * GB + bi) * K, K), :] = jnp.dot(
            oh_t, xc1, preferred_element_type=jnp.float32)

    @pl.when(g == B // GB - 1)
    def _tail():
        sums = seg[:, :INTER]
        cnt = seg[:, INTER:INTER + 1]
        mean = sums / cnt
        h2 = _leaky(jnp.dot(mean, w3[...], preferred_element_type=jnp.float32) + b3[...])
        xsp = jnp.dot(h2, w4[...], preferred_element_type=jnp.float32) + b4[...]
        xsp = jnp.where(cnt > 0, xsp, 0.0)
        ohb = (lax.broadcasted_iota(jnp.int32, (B, B * K), 1) // K
               == lax.broadcasted_iota(jnp.int32, (B, B * K), 0)).astype(jnp.float32)
        p = jnp.dot(ohb, xsp, preferred_element_type=jnp.float32)
        t = _leaky(jnp.dot(p, w5[...], preferred_element_type=jnp.float32) + b5[...])
        logits = jnp.dot(t, w6[...], preferred_element_type=jnp.float32) + b6[...]
        m = jnp.max(logits, axis=-1, keepdims=True)
        lse = jnp.log(jnp.sum(jnp.exp(logits - m), axis=-1, keepdims=True)) + m
        y_ref[...] = logits - lse


def kernel(x, W1, b1, W2, b2, W3, b3, W4, b4, W5, b5, W6, b6, centroids):
    full = lambda shape: pl.BlockSpec(shape, lambda b: (0,) * len(shape))
    b1r, b2r, b3r, b4r = (v.reshape(1, -1) for v in (b1, b2, b3, b4))
    b5r, b6r = b5.reshape(1, -1), b6.reshape(1, -1)

    assign, y_pred = pl.pallas_call(
        _fused_body,
        grid=(B // GB,),
        in_specs=[
            pl.BlockSpec((GB, P, D), lambda b: (b, 0, 0)),
            full((D, 64)), full((1, 64)),
            full((64, INTER)), full((1, INTER)),
            full((INTER, 64)), full((1, 64)),
            full((64, POOL)), full((1, POOL)),
            full((POOL, 64)), full((1, 64)),
            full((64, OUT)), full((1, OUT)),
            full((K, INTER)),
        ],
        out_specs=[
            pl.BlockSpec((GB, 1, P), lambda b: (b, 0, 0)),
            pl.BlockSpec((B, OUT), lambda b: (0, 0)),
        ],
        out_shape=[
            jax.ShapeDtypeStruct((B, 1, P), jnp.int32),
            jax.ShapeDtypeStruct((B, OUT), jnp.float32),
        ],
        scratch_shapes=[pltpu.VMEM((B * K, INTER + 1), jnp.float32)],
    )(x, W1, b1r, W2, b2r, W3, b3r, W4, b4r, W5, b5r, W6, b6r, centroids)

    return (y_pred, assign.reshape(B * P))


# speedup vs baseline: 1.0668x; 1.0668x over previous
import functools

import jax
import jax.numpy as jnp
from jax import lax
from jax.experimental import pallas as pl
from jax.experimental.pallas import tpu as pltpu

B, P, D = 16, 4096, 128
INTER, POOL, OUT = 64, 64, 32
K = 16
GB = 2

_SLOPE = 0.01


def _leaky(v):
    return jnp.where(v >= 0, v, _SLOPE * v)


def _fused_body(x_ref, w1, b1, w2, b2, w3, b3, w4, b4, w5, b5, w6, b6, cen,
                assign_ref, y_ref, seg):
    g = pl.program_id(0)
    c = cen[...]
    cn = jnp.sum(c * c, axis=1, keepdims=True)

    PP = GB * P
    xb = x_ref[...].reshape(PP, D)
    h = _leaky(jnp.dot(xb, w1[...], preferred_element_type=jnp.float32) + b1[...])
    xc = jnp.dot(h, w2[...], preferred_element_type=jnp.float32) + b2[...]
    cross_t = lax.dot_general(c, xc, (((1,), (1,)), ((), ())),
                              preferred_element_type=jnp.float32)
    score = cn - 2.0 * cross_t
    best = jnp.min(score, axis=0, keepdims=True)
    ii = lax.broadcasted_iota(jnp.int32, (K, PP), 0)
    a2d = jnp.min(jnp.where(score == best, ii, K), axis=0, keepdims=True)

    oh_t = (ii == a2d).astype(jnp.float32)
    xc1 = jnp.concatenate([xc, jnp.ones((PP, 1), jnp.float32)], axis=1)
    for bi in range(GB):
        assign_ref[bi] = a2d[:, bi * P:(bi + 1) * P]
        seg[pl.ds((g * GB + bi) * K, K), :] = jnp.dot(
            oh_t[:, bi * P:(bi + 1) * P], xc1[bi * P:(bi + 1) * P],
            preferred_element_type=jnp.float32)

    @pl.when(g == B // GB - 1)
    def _tail():
        sums = seg[:, :INTER]
        cnt = seg[:, INTER:INTER + 1]
        mean = sums / cnt
        h2 = _leaky(jnp.dot(mean, w3[...], preferred_element_type=jnp.float32) + b3[...])
        xsp = jnp.dot(h2, w4[...], preferred_element_type=jnp.float32) + b4[...]
        xsp = jnp.where(cnt > 0, xsp, 0.0)
        ohb = (lax.broadcasted_iota(jnp.int32, (B, B * K), 1) // K
               == lax.broadcasted_iota(jnp.int32, (B, B * K), 0)).astype(jnp.float32)
        p = jnp.dot(ohb, xsp, preferred_element_type=jnp.float32)
        t = _leaky(jnp.dot(p, w5[...], preferred_element_type=jnp.float32) + b5[...])
        logits = jnp.dot(t, w6[...], preferred_element_type=jnp.float32) + b6[...]
        m = jnp.max(logits, axis=-1, keepdims=True)
        lse = jnp.log(jnp.sum(jnp.exp(logits - m), axis=-1, keepdims=True)) + m
        y_ref[...] = logits - lse


def kernel(x, W1, b1, W2, b2, W3, b3, W4, b4, W5, b5, W6, b6, centroids):
    full = lambda shape: pl.BlockSpec(shape, lambda b: (0,) * len(shape))
    b1r, b2r, b3r, b4r = (v.reshape(1, -1) for v in (b1, b2, b3, b4))
    b5r, b6r = b5.reshape(1, -1), b6.reshape(1, -1)

    assign, y_pred = pl.pallas_call(
        _fused_body,
        grid=(B // GB,),
        in_specs=[
            pl.BlockSpec((GB, P, D), lambda b: (b, 0, 0)),
            full((D, 64)), full((1, 64)),
            full((64, INTER)), full((1, INTER)),
            full((INTER, 64)), full((1, 64)),
            full((64, POOL)), full((1, POOL)),
            full((POOL, 64)), full((1, 64)),
            full((64, OUT)), full((1, OUT)),
            full((K, INTER)),
        ],
        out_specs=[
            pl.BlockSpec((GB, 1, P), lambda b: (b, 0, 0)),
            pl.BlockSpec((B, OUT), lambda b: (0, 0)),
        ],
        out_shape=[
            jax.ShapeDtypeStruct((B, 1, P), jnp.int32),
            jax.ShapeDtypeStruct((B, OUT), jnp.float32),
        ],
        scratch_shapes=[pltpu.VMEM((B * K, INTER + 1), jnp.float32)],
    )(x, W1, b1r, W2, b2r, W3, b3r, W4, b4r, W5, b5r, W6, b6r, centroids)

    return (y_pred, assign.reshape(B * P))


# speedup vs baseline: 1.0681x; 1.0012x over previous
import functools

import jax
import jax.numpy as jnp
from jax import lax
from jax.experimental import pallas as pl
from jax.experimental.pallas import tpu as pltpu

B, P, D = 16, 4096, 128
INTER, POOL, OUT = 64, 64, 32
K = 16
GB = 2

_SLOPE = 0.01


def _leaky(v):
    return jnp.maximum(v, _SLOPE * v)


def _fused_body(x_ref, w1, b1, w2, b2, w3, b3, w4, b4, w5, b5, w6, b6, cen,
                assign_ref, y_ref, seg):
    g = pl.program_id(0)
    c = cen[...]
    cn = jnp.sum(c * c, axis=1, keepdims=True)

    PP = GB * P
    xb = x_ref[...].reshape(PP, D)
    h = _leaky(jnp.dot(xb, w1[...], preferred_element_type=jnp.float32) + b1[...])
    xc = jnp.dot(h, w2[...], preferred_element_type=jnp.float32) + b2[...]
    xc1 = jnp.concatenate([xc, jnp.ones((PP, 1), jnp.float32)], axis=1)
    c2cn = jnp.concatenate([-2.0 * c, cn], axis=1)
    score = lax.dot_general(c2cn, xc1, (((1,), (1,)), ((), ())),
                            preferred_element_type=jnp.float32)
    best = jnp.min(score, axis=0, keepdims=True)
    ii = lax.broadcasted_iota(jnp.int32, (K, PP), 0)
    a2d = jnp.min(jnp.where(score == best, ii, K), axis=0, keepdims=True)

    oh_t = (ii == a2d).astype(jnp.float32)
    for bi in range(GB):
        assign_ref[bi] = a2d[:, bi * P:(bi + 1) * P]
        seg[pl.ds((g * GB + bi) * K, K), :] = jnp.dot(
            oh_t[:, bi * P:(bi + 1) * P], xc1[bi * P:(bi + 1) * P],
            preferred_element_type=jnp.float32)

    @pl.when(g == B // GB - 1)
    def _tail():
        sums = seg[:, :INTER]
        cnt = seg[:, INTER:INTER + 1]
        mean = sums / cnt
        h2 = _leaky(jnp.dot(mean, w3[...], preferred_element_type=jnp.float32) + b3[...])
        xsp = jnp.dot(h2, w4[...], preferred_element_type=jnp.float32) + b4[...]
        xsp = jnp.where(cnt > 0, xsp, 0.0)
        ohb = (lax.broadcasted_iota(jnp.int32, (B, B * K), 1) // K
               == lax.broadcasted_iota(jnp.int32, (B, B * K), 0)).astype(jnp.float32)
        p = jnp.dot(ohb, xsp, preferred_element_type=jnp.float32)
        t = _leaky(jnp.dot(p, w5[...], preferred_element_type=jnp.float32) + b5[...])
        logits = jnp.dot(t, w6[...], preferred_element_type=jnp.float32) + b6[...]
        m = jnp.max(logits, axis=-1, keepdims=True)
        lse = jnp.log(jnp.sum(jnp.exp(logits - m), axis=-1, keepdims=True)) + m
        y_ref[...] = logits - lse


def kernel(x, W1, b1, W2, b2, W3, b3, W4, b4, W5, b5, W6, b6, centroids):
    full = lambda shape: pl.BlockSpec(shape, lambda b: (0,) * len(shape))
    b1r, b2r, b3r, b4r = (v.reshape(1, -1) for v in (b1, b2, b3, b4))
    b5r, b6r = b5.reshape(1, -1), b6.reshape(1, -1)

    assign, y_pred = pl.pallas_call(
        _fused_body,
        grid=(B // GB,),
        in_specs=[
            pl.BlockSpec((GB, P, D), lambda b: (b, 0, 0)),
            full((D, 64)), full((1, 64)),
            full((64, INTER)), full((1, INTER)),
            full((INTER, 64)), full((1, 64)),
            full((64, POOL)), full((1, POOL)),
            full((POOL, 64)), full((1, 64)),
            full((64, OUT)), full((1, OUT)),
            full((K, INTER)),
        ],
        out_specs=[
            pl.BlockSpec((GB, 1, P), lambda b: (b, 0, 0)),
            pl.BlockSpec((B, OUT), lambda b: (0, 0)),
        ],
        out_shape=[
            jax.ShapeDtypeStruct((B, 1, P), jnp.int32),
            jax.ShapeDtypeStruct((B, OUT), jnp.float32),
        ],
        scratch_shapes=[pltpu.VMEM((B * K, INTER + 1), jnp.float32)],
    )(x, W1, b1r, W2, b2r, W3, b3r, W4, b4r, W5, b5r, W6, b6r, centroids)

    return (y_pred, assign.reshape(B * P))
